# Initial kernel scaffold; baseline (speedup 1.0000x reference)
import jax
import jax.numpy as jnp
from jax.experimental import pallas as pl


def kernel(x, edge_index, edge_attr):
    raise NotImplementedError("write your pallas kernel here")


# speedup vs baseline: 2.3150x; 2.3150x over previous
import functools

import jax
import jax.numpy as jnp
from jax import lax
from jax.experimental import pallas as pl
from jax.experimental.pallas import tpu as pltpu
from jax.experimental.pallas import tpu_sc as plsc

NC = 2
NS = 16
L = 16
NW = NC * NS

C = 80

_GD = lax.GatherDimensionNumbers(
    offset_dims=(), collapsed_slice_dims=(0,), start_index_map=(0,))


def _lanes(x, idx):
    return lax.gather(x, idx[:, None], _GD, (1,),
                      mode=lax.GatherScatterMode.PROMISE_IN_BOUNDS)


def _make_agg(n, d, e):
    ew = e // NW
    nch = ew // C
    nrc = n // C
    rpw = (nrc + NS - 1) // NS
    mesh = plsc.VectorSubcoreMesh(core_axis_name="c", subcore_axis_name="s")

    @functools.partial(
        pl.kernel,
        out_type=jax.ShapeDtypeStruct((NC, n, d), jnp.float32),
        mesh=mesh,
        scratch_types=[
            pltpu.VMEM_SHARED((n, d), jnp.float32),
            pltpu.VMEM((C,), jnp.int32),
            pltpu.VMEM((C,), jnp.int32),
            pltpu.VMEM((C,), jnp.float32),
            pltpu.VMEM((C, d), jnp.float32),
            pltpu.VMEM((C, d), jnp.float32),
            pltpu.SemaphoreType.DMA,
        ],
    )
    def agg(x_hbm, row_hbm, col_hbm, w_hbm, out_hbm,
            agg_sh, rowv, colv, wv, xr, zb, sem):
        c = lax.axis_index("c")
        s = lax.axis_index("s")
        wid = s * NC + c
        lane = lax.iota(jnp.int32, L)

        zv = jnp.zeros((L,), jnp.float32)
        for i in range(C):
            for k in range(d // L):
                zb[i, pl.ds(k * L, L)] = zv

        for i in range(rpw):
            cid = i * NS + s

            @pl.when(cid < nrc)
            def _():
                pltpu.sync_copy(zb, agg_sh.at[pl.ds(cid * C, C)])

        plsc.subcore_barrier()

        e0 = wid * ew

        def body(i, carry):
            base = e0 + i * C
            pltpu.sync_copy(row_hbm.at[pl.ds(base, C)], rowv)
            pltpu.sync_copy(col_hbm.at[pl.ds(base, C)], colv)
            pltpu.sync_copy(w_hbm.at[pl.ds(base, C)], wv)
            pltpu.async_copy(x_hbm.at[rowv], xr, sem).wait()
            for j in range(C // L):
                w16 = wv[pl.ds(j * L, L)]
                for jj in range(L):
                    ei = j * L + jj
                    ws = _lanes(w16, jnp.full((L,), jj, jnp.int32))
                    for k in range(d // L):
                        sl = pl.ds(k * L, L)
                        xr[ei, sl] = xr[ei, sl] * ws
            pltpu.sync_copy(xr, agg_sh.at[colv], add=True)
            return carry

        lax.fori_loop(0, nch, body, 0)
        plsc.subcore_barrier()

        for i in range(rpw):
            cid = i * NS + s

            @pl.when(cid < nrc)
            def _():
                sl = pl.ds(cid * C, C)
                pltpu.sync_copy(agg_sh.at[sl], zb)
                pltpu.sync_copy(zb, out_hbm.at[c, sl])

    return agg


def _make_deg(n, e):
    dw = 128
    ew = e // NW
    nch = ew // C
    nrc = n // C
    rpw = (nrc + NS - 1) // NS
    mesh = plsc.VectorSubcoreMesh(core_axis_name="c", subcore_axis_name="s")

    @functools.partial(
        pl.kernel,
        out_type=jax.ShapeDtypeStruct((NC, n, dw), jnp.float32),
        mesh=mesh,
        scratch_types=[
            pltpu.VMEM_SHARED((n, dw), jnp.float32),
            pltpu.VMEM((C,), jnp.int32),
            pltpu.VMEM((C, dw), jnp.float32),
            pltpu.VMEM((C, dw), jnp.float32),
        ],
    )
    def deg(col_hbm, out_hbm, deg_sh, colv, ones_v, zb):
        c = lax.axis_index("c")
        s = lax.axis_index("s")
        wid = s * NC + c

        ov = jnp.ones((L,), jnp.float32)
        zv = jnp.zeros((L,), jnp.float32)
        for i in range(C):
            for k in range(dw // L):
                ones_v[i, pl.ds(k * L, L)] = ov
                zb[i, pl.ds(k * L, L)] = zv

        for i in range(rpw):
            cid = i * NS + s

            @pl.when(cid < nrc)
            def _():
                pltpu.sync_copy(zb, deg_sh.at[pl.ds(cid * C, C)])

        plsc.subcore_barrier()

        e0 = wid * ew

        def body(i, carry):
            base = e0 + i * C
            pltpu.sync_copy(col_hbm.at[pl.ds(base, C)], colv)
            pltpu.sync_copy(ones_v, deg_sh.at[colv], add=True)
            return carry

        lax.fori_loop(0, nch, body, 0)
        plsc.subcore_barrier()

        for i in range(rpw):
            cid = i * NS + s

            @pl.when(cid < nrc)
            def _():
                sl = pl.ds(cid * C, C)
                pltpu.sync_copy(deg_sh.at[sl], zb)
                pltpu.sync_copy(zb, out_hbm.at[c, sl])

    return deg


def _make_dist(n, d, e):
    ew = e // NW
    nch = ew // C
    mesh = plsc.VectorSubcoreMesh(core_axis_name="c", subcore_axis_name="s")

    @functools.partial(
        pl.kernel,
        out_type=jax.ShapeDtypeStruct((e,), jnp.float32),
        mesh=mesh,
        scratch_types=[
            pltpu.VMEM((C,), jnp.int32),
            pltpu.VMEM((C,), jnp.int32),
            pltpu.VMEM((C, d), jnp.float32),
            pltpu.VMEM((C, d), jnp.float32),
            pltpu.VMEM((C,), jnp.float32),
            pltpu.SemaphoreType.DMA,
            pltpu.SemaphoreType.DMA,
        ],
    )
    def dist(x_hbm, row_hbm, col_hbm, s_hbm,
             rowv, colv, xr, xc, sv, sem1, sem2):
        c = lax.axis_index("c")
        s = lax.axis_index("s")
        wid = s * NC + c
        lane = lax.iota(jnp.int32, L)
        e0 = wid * ew

        def body(i, carry):
            base = e0 + i * C
            pltpu.sync_copy(row_hbm.at[pl.ds(base, C)], rowv)
            pltpu.sync_copy(col_hbm.at[pl.ds(base, C)], colv)
            cp1 = pltpu.async_copy(x_hbm.at[rowv], xr, sem1)
            cp2 = pltpu.async_copy(x_hbm.at[colv], xc, sem2)
            cp1.wait()
            cp2.wait()
            perms = [jnp.bitwise_and(lane + (1 << p), L - 1)
                     for p in range(4)]
            for j in range(C // L):
                s16 = jnp.zeros((L,), jnp.float32)
                for jj in range(L):
                    ei = j * L + jj
                    acc = jnp.zeros((L,), jnp.float32)
                    for k in range(d // L):
                        sl = pl.ds(k * L, L)
                        dv = xr[ei, sl] - xc[ei, sl]
                        acc = acc + dv * dv
                    for p in perms:
                        acc = acc + _lanes(acc, p)
                    s16 = jnp.where(lane == jj, acc, s16)
                sv[pl.ds(j * L, L)] = s16
            pltpu.sync_copy(sv, s_hbm.at[pl.ds(base, C)])
            return carry

        lax.fori_loop(0, nch, body, 0)

    return dist


def _w0_tc(edge_attr):
    e, a = edge_attr.shape
    ea_t = edge_attr.T.reshape(a, e // 128, 128)

    def body(a_ref, o_ref):
        o_ref[...] = jnp.sqrt(jnp.sum(a_ref[...] ** 2, axis=0))

    out = pl.pallas_call(
        body, out_shape=jax.ShapeDtypeStruct((e // 128, 128), jnp.float32),
    )(ea_t)
    return out.reshape(e)


def _xupd_tc(x, agg0, agg1, degsum):
    n, d = x.shape
    blk = 2000

    def body(x_ref, a0_ref, a1_ref, d_ref, o_ref):
        dd = jnp.maximum(d_ref[...], 1.0)
        o_ref[...] = 0.5 * (x_ref[...] + (a0_ref[...] + a1_ref[...]) / dd)

    bs = pl.BlockSpec((blk, d), lambda i: (i, 0))
    bd = pl.BlockSpec((blk, 1), lambda i: (i, 0))
    return pl.pallas_call(
        body,
        grid=(n // blk,),
        in_specs=[bs, bs, bs, bd],
        out_specs=bs,
        out_shape=jax.ShapeDtypeStruct((n, d), jnp.float32),
    )(x, agg0, agg1, degsum)


def _wupd_tc(sq):
    e = sq.shape[0]
    s2 = sq.reshape(e // 128, 128)

    def body(s_ref, o_ref):
        m = jnp.max(s_ref[...])
        o_ref[...] = jnp.sqrt(s_ref[...] / m)

    out = pl.pallas_call(
        body, out_shape=jax.ShapeDtypeStruct(s2.shape, jnp.float32),
    )(s2)
    return out.reshape(e)


def kernel(x, edge_index, edge_attr):
    n, d = x.shape
    e = edge_index.shape[1]
    row = edge_index[0].astype(jnp.int32)
    col = edge_index[1].astype(jnp.int32)

    w = _w0_tc(edge_attr)
    deg_parts = _make_deg(n, e)(col)
    degsum = deg_parts[0, :, :1] + deg_parts[1, :, :1]

    aggf = _make_agg(n, d, e)
    distf = _make_dist(n, d, e)

    feats = []
    for it in range(3):
        parts = aggf(x, row, col, w)
        x = _xupd_tc(x, parts[0], parts[1], degsum)
        feats.append(x)
        if it < 2:
            sq = distf(x, row, col)
            w = _wupd_tc(sq)
    return jnp.concatenate(feats, axis=-1)
